# Initial kernel scaffold; baseline (speedup 1.0000x reference)
import jax
import jax.numpy as jnp
from jax.experimental import pallas as pl


def kernel(input, aux, device):
    raise NotImplementedError("write your pallas kernel here")


# speedup vs baseline: 93.8744x; 93.8744x over previous
from functools import partial, reduce

import jax
import jax.numpy as jnp
from jax.experimental import pallas as pl
from jax.experimental.pallas import tpu as pltpu

_PASS_DIRS = (
    (1, 0, 0), (0, 1, 0),
    (1, 0, 0), (0, 0, 1),
    (0, 1, 0), (0, 0, 1),
    (1, 0, 0), (0, 1, 1),
    (1, 0, 0), (0, 1, -1),
    (0, 1, 0), (1, 0, 1),
    (0, 1, 0), (1, 0, -1),
    (0, 0, 1), (1, 1, 0),
    (0, 0, 1), (1, -1, 0),
)

_U32 = jnp.uint32


def _pack_kernel(inp_ref, aux_ref, out_ref):
    g = pl.program_id(0) % 2
    inp = inp_ref[0]
    aux = aux_ref[0]
    b = (aux < 0) | ((aux == 0) & (inp != 0))
    x0 = jnp.logical_xor(b, g == 1)
    d_iota = jax.lax.broadcasted_iota(_U32, x0.shape, 0)
    bits = x0.astype(_U32) << d_iota
    word = reduce(jnp.bitwise_or, [bits[j] for j in range(bits.shape[0])])
    out_ref[0, 0] = word


def _fill_slab(shape, fw):
    return jnp.full(shape, fw, _U32)


def _shift_h(x, dh, fw):
    if dh == 0:
        return x
    pad = _fill_slab(x.shape[:1] + (abs(dh),) + x.shape[2:], fw)
    if dh > 0:
        return jnp.concatenate([x[:, dh:], pad], axis=1)
    return jnp.concatenate([pad, x[:, :dh]], axis=1)


def _shift_w(x, dw, fw):
    if dw == 0:
        return x
    pad = _fill_slab(x.shape[:-1] + (abs(dw),), fw)
    if dw > 0:
        return jnp.concatenate([x[..., dw:], pad], axis=-1)
    return jnp.concatenate([pad, x[..., :dw]], axis=-1)


def _shift_d(x, dd, fw):
    if dd == 0:
        return x
    pad = _fill_slab((1,) + x.shape[1:], fw)
    if dd > 0:
        nxt = jnp.concatenate([x[1:], pad], axis=0)
        return (x >> 1) | (nxt << 31)
    prv = jnp.concatenate([pad, x[:-1]], axis=0)
    return (x << 1) | (prv >> 31)


def _tap(x, dd, dh, dw, fw):
    y = _shift_d(x, dd, fw)
    y = _shift_h(y, dh, fw)
    return _shift_w(y, dw, fw)


def _chain(x, fw):
    for dd, dh, dw in _PASS_DIRS:
        x = x | _tap(x, dd, dh, dw, fw) | _tap(x, -dd, -dh, -dw, fw)
    return x


def _morph_kernel(x_ref, o_ref):
    gi = pl.program_id(0) % 2
    fw1 = (0 - gi).astype(_U32)
    fw2 = (gi - 1).astype(_U32)
    x = x_ref[0]
    z = _chain(x, fw1)
    t = _chain(~z, fw2)
    o_ref[0] = t ^ fw2


def _unpack_kernel(x_ref, o_ref):
    word = x_ref[0, 0]
    nd = o_ref.shape[1]
    one = jnp.uint32(1)
    out = jnp.stack(
        [((word >> jnp.uint32(j)) & one).astype(jnp.float32)
         for j in range(nd)], axis=0)
    o_ref[0] = out


def kernel(input, aux, device):
    del device
    B, C, D, H, W = input.shape
    ND = D // 32
    inp4 = input.reshape(B, D, H, W)
    aux4 = aux.reshape(B, D, H, W)

    packed = pl.pallas_call(
        _pack_kernel,
        grid=(B, ND),
        in_specs=[
            pl.BlockSpec((1, 32, H, W), lambda i, k: (i, k, 0, 0)),
            pl.BlockSpec((1, 32, H, W), lambda i, k: (i, k, 0, 0)),
        ],
        out_specs=pl.BlockSpec((1, 1, H, W), lambda i, k: (i, k, 0, 0)),
        out_shape=jax.ShapeDtypeStruct((B, ND, H, W), _U32),
        compiler_params=pltpu.CompilerParams(
            dimension_semantics=("parallel", "arbitrary"),
        ),
        name="morph_pack",
    )(inp4, aux4)

    morphed = pl.pallas_call(
        _morph_kernel,
        grid=(B,),
        in_specs=[pl.BlockSpec((1, ND, H, W), lambda i: (i, 0, 0, 0))],
        out_specs=pl.BlockSpec((1, ND, H, W), lambda i: (i, 0, 0, 0)),
        out_shape=jax.ShapeDtypeStruct((B, ND, H, W), _U32),
        compiler_params=pltpu.CompilerParams(
            dimension_semantics=("arbitrary",),
            vmem_limit_bytes=48 * 1024 * 1024,
        ),
        name="morph_chain_packed",
    )(packed)

    out = pl.pallas_call(
        _unpack_kernel,
        grid=(B, ND),
        in_specs=[pl.BlockSpec((1, 1, H, W), lambda i, k: (i, k, 0, 0))],
        out_specs=pl.BlockSpec((1, 32, H, W), lambda i, k: (i, k, 0, 0)),
        out_shape=jax.ShapeDtypeStruct((B, D, H, W), jnp.float32),
        compiler_params=pltpu.CompilerParams(
            dimension_semantics=("parallel", "arbitrary"),
        ),
        name="morph_unpack",
    )(morphed)

    return out.reshape(B, C, D, H, W)


# speedup vs baseline: 95.6148x; 1.0185x over previous
from functools import partial, reduce

import jax
import jax.numpy as jnp
from jax.experimental import pallas as pl
from jax.experimental.pallas import tpu as pltpu

_PASS_DIRS = (
    (1, 0, 0), (0, 1, 0),
    (1, 0, 0), (0, 0, 1),
    (0, 1, 0), (0, 0, 1),
    (1, 0, 0), (0, 1, 1),
    (1, 0, 0), (0, 1, -1),
    (0, 1, 0), (1, 0, 1),
    (0, 1, 0), (1, 0, -1),
    (0, 0, 1), (1, 1, 0),
    (0, 0, 1), (1, -1, 0),
)

_U32 = jnp.uint32


def _pack_morph_kernel(inp_ref, aux_ref, o_ref):
    gi = pl.program_id(0) % 2
    k = pl.program_id(1)
    nd = o_ref.shape[1]
    inp = inp_ref[0]
    aux = aux_ref[0]
    b = (aux < 0) | ((aux == 0) & (inp != 0))
    x0 = jnp.logical_xor(b, gi == 1)
    d_iota = jax.lax.broadcasted_iota(_U32, x0.shape, 0)
    bits = x0.astype(_U32) << d_iota
    word = reduce(jnp.bitwise_or, [bits[j] for j in range(bits.shape[0])])
    o_ref[0, k] = word

    @pl.when(k == nd - 1)
    def _():
        fw1 = (0 - gi).astype(_U32)
        fw2 = (gi - 1).astype(_U32)
        x = o_ref[0]
        z = _chain(x, fw1)
        t = _chain(~z, fw2)
        o_ref[0] = t ^ fw2


def _fill_slab(shape, fw):
    return jnp.full(shape, fw, _U32)


def _shift_h(x, dh, fw):
    if dh == 0:
        return x
    pad = _fill_slab(x.shape[:1] + (abs(dh),) + x.shape[2:], fw)
    if dh > 0:
        return jnp.concatenate([x[:, dh:], pad], axis=1)
    return jnp.concatenate([pad, x[:, :dh]], axis=1)


def _shift_w(x, dw, fw):
    if dw == 0:
        return x
    pad = _fill_slab(x.shape[:-1] + (abs(dw),), fw)
    if dw > 0:
        return jnp.concatenate([x[..., dw:], pad], axis=-1)
    return jnp.concatenate([pad, x[..., :dw]], axis=-1)


def _shift_d(x, dd, fw):
    if dd == 0:
        return x
    pad = _fill_slab((1,) + x.shape[1:], fw)
    if dd > 0:
        nxt = jnp.concatenate([x[1:], pad], axis=0)
        return (x >> 1) | (nxt << 31)
    prv = jnp.concatenate([pad, x[:-1]], axis=0)
    return (x << 1) | (prv >> 31)


def _tap(x, dd, dh, dw, fw):
    y = _shift_d(x, dd, fw)
    y = _shift_h(y, dh, fw)
    return _shift_w(y, dw, fw)


def _chain(x, fw):
    for dd, dh, dw in _PASS_DIRS:
        x = x | _tap(x, dd, dh, dw, fw) | _tap(x, -dd, -dh, -dw, fw)
    return x


def _unpack_kernel(x_ref, o_ref):
    word = x_ref[0, 0]
    nd = o_ref.shape[1]
    one = jnp.uint32(1)
    out = jnp.stack(
        [((word >> jnp.uint32(j)) & one).astype(jnp.float32)
         for j in range(nd)], axis=0)
    o_ref[0] = out


def kernel(input, aux, device):
    del device
    B, C, D, H, W = input.shape
    ND = D // 32
    inp4 = input.reshape(B, D, H, W)
    aux4 = aux.reshape(B, D, H, W)

    morphed = pl.pallas_call(
        _pack_morph_kernel,
        grid=(B, ND),
        in_specs=[
            pl.BlockSpec((1, 32, H, W), lambda i, k: (i, k, 0, 0)),
            pl.BlockSpec((1, 32, H, W), lambda i, k: (i, k, 0, 0)),
        ],
        out_specs=pl.BlockSpec((1, ND, H, W), lambda i, k: (i, 0, 0, 0)),
        out_shape=jax.ShapeDtypeStruct((B, ND, H, W), _U32),
        compiler_params=pltpu.CompilerParams(
            dimension_semantics=("parallel", "arbitrary"),
            vmem_limit_bytes=48 * 1024 * 1024,
        ),
        name="morph_pack_chain",
    )(inp4, aux4)

    out = pl.pallas_call(
        _unpack_kernel,
        grid=(B, ND),
        in_specs=[pl.BlockSpec((1, 1, H, W), lambda i, k: (i, k, 0, 0))],
        out_specs=pl.BlockSpec((1, 32, H, W), lambda i, k: (i, k, 0, 0)),
        out_shape=jax.ShapeDtypeStruct((B, D, H, W), jnp.float32),
        compiler_params=pltpu.CompilerParams(
            dimension_semantics=("parallel", "arbitrary"),
        ),
        name="morph_unpack",
    )(morphed)

    return out.reshape(B, C, D, H, W)
